# baseline (speedup 1.0000x reference)
import functools

import jax
import jax.numpy as jnp
from jax import lax
from jax.experimental import pallas as pl
from jax.experimental.pallas import tpu as pltpu
from jax.experimental.pallas import tpu_sc as plsc

N = 10000
NP = 10240
E = 320000
NC = 2
NS = 16
NW = NC * NS
CH = 128
EPAD = 327680
EPW = EPAD // NW
NCH = EPW // CH
RPT = NP // NS
ICH = 128

_MESH = plsc.VectorSubcoreMesh(core_axis_name="c", subcore_axis_name="s")


def _make_deg():
    @functools.partial(
        pl.kernel,
        out_type=[
            jax.ShapeDtypeStruct((NP,), jnp.float32),
            jax.ShapeDtypeStruct((NP,), jnp.float32),
        ],
        mesh=_MESH,
        scratch_types=[
            pltpu.VMEM((NCH, CH), jnp.int32),
            pltpu.VMEM((RPT,), jnp.float32),
            pltpu.VMEM((CH,), jnp.float32),
            pltpu.VMEM_SHARED((NP,), jnp.float32),
        ] + [pltpu.SemaphoreType.DMA] * 8,
    )
    def deg_kernel(dst_hbm, out0, out1, dst_v, zb_v, ones_v, acc_sh, *sems):
        cid = lax.axis_index("c")
        sid = lax.axis_index("s")
        w = sid * NC + cid
        r0 = sid * RPT

        pltpu.sync_copy(dst_hbm.at[pl.ds(w * NCH, NCH)], dst_v)

        def zfill(i, _):
            zb_v[pl.ds(i * 16, 16)] = jnp.zeros((16,), jnp.float32)
            return _
        lax.fori_loop(0, RPT // 16, zfill, 0)
        pltpu.sync_copy(zb_v, acc_sh.at[pl.ds(r0, RPT)])

        def ofill(i, _):
            ones_v[pl.ds(i * 16, 16)] = jnp.ones((16,), jnp.float32)
            return _
        lax.fori_loop(0, CH // 16, ofill, 0)
        plsc.subcore_barrier()

        NSEM = 8

        def body(jj, carry):
            j = jj * NSEM
            for b in range(NSEM):
                @pl.when(jj > 0)
                def _drain():
                    pltpu.make_async_copy(
                        ones_v, acc_sh.at[dst_v.at[0]], sems[b]).wait()
                pltpu.async_copy(ones_v, acc_sh.at[dst_v.at[j + b]],
                                 sems[b], add=True)
            return carry
        lax.fori_loop(0, NCH // NSEM, body, 0)
        for b in range(NSEM):
            pltpu.make_async_copy(ones_v, acc_sh.at[dst_v.at[0]],
                                  sems[b]).wait()
        plsc.subcore_barrier()

        pltpu.sync_copy(acc_sh.at[pl.ds(r0, RPT)], zb_v)

        @pl.when(cid == 0)
        def _():
            pltpu.sync_copy(zb_v, out0.at[pl.ds(r0, RPT)])

        @pl.when(cid == 1)
        def _():
            pltpu.sync_copy(zb_v, out1.at[pl.ds(r0, RPT)])

    return deg_kernel


def _make_prop(F, nbuf):
    @functools.partial(
        pl.kernel,
        out_type=[
            jax.ShapeDtypeStruct((NP, F), jnp.float32),
            jax.ShapeDtypeStruct((NP, F), jnp.float32),
        ],
        mesh=_MESH,
        compiler_params=pltpu.CompilerParams(
            use_tc_tiling_on_sc=False) if F < 128 else None,
        scratch_types=[
            pltpu.VMEM((NCH // 2, CH), jnp.int32),
            pltpu.VMEM((NCH // 2, CH), jnp.int32),
            pltpu.VMEM_SHARED((NP, F), jnp.float32),
        ] + [pltpu.VMEM((CH, F), jnp.float32)] * nbuf
          + [pltpu.SemaphoreType.DMA] * nbuf,
    )
    def prop_kernel(hp_hbm, src_hbm, dst_hbm, out0, out1,
                    src_v, dst_v, acc_sh, *scr):
        rows = scr[:nbuf]
        sems = scr[nbuf:]
        cid = lax.axis_index("c")
        sid = lax.axis_index("s")
        w = sid * NC + cid
        r0 = sid * RPT
        HN = NCH // 2

        for t in range(RPT // ICH):
            r = r0 + t * ICH
            pltpu.sync_copy(hp_hbm.at[pl.ds(r, ICH)], rows[0])
            pltpu.sync_copy(rows[0], acc_sh.at[pl.ds(r, ICH)])
        plsc.subcore_barrier()

        for h in range(2):
            pltpu.sync_copy(src_hbm.at[pl.ds(w * NCH + h * HN, HN)], src_v)
            pltpu.sync_copy(dst_hbm.at[pl.ds(w * NCH + h * HN, HN)], dst_v)
            for b in range(nbuf - 1):
                pltpu.async_copy(hp_hbm.at[src_v.at[b]], rows[b], sems[b])

            def body(jj, carry):
                j = jj * nbuf
                for b in range(nbuf):
                    p = j + b + nbuf - 1
                    pb = (b + nbuf - 1) % nbuf

                    @pl.when(p < HN)
                    def _issue_next():
                        pltpu.async_copy(
                            hp_hbm.at[src_v.at[p]], rows[pb], sems[pb])
                    pltpu.make_async_copy(
                        hp_hbm.at[src_v.at[0]], rows[b], sems[b]).wait()
                    pltpu.sync_copy(rows[b], acc_sh.at[dst_v.at[j + b]],
                                    add=True)
                return carry
            lax.fori_loop(0, HN // nbuf, body, 0)
        plsc.subcore_barrier()

        for t in range(RPT // ICH):
            r = r0 + t * ICH
            pltpu.sync_copy(acc_sh.at[pl.ds(r, ICH)], rows[0])

            @pl.when(cid == 0)
            def _():
                pltpu.sync_copy(rows[0], out0.at[pl.ds(r, ICH)])

            @pl.when(cid == 1)
            def _():
                pltpu.sync_copy(rows[0], out1.at[pl.ds(r, ICH)])

    return prop_kernel


_deg = _make_deg()
_prop128 = _make_prop(128, 2)
_prop64 = _make_prop(64, 4)


R = 2048


def _row_spec(f):
    return pl.BlockSpec((R, f), lambda i: (i, 0))


def _full_spec(shape):
    return pl.BlockSpec(shape, lambda i: tuple(0 for _ in shape))


def _scale0_body(d0_ref, d1_ref, x_ref, dinv_ref, hp0_ref):
    dinv = lax.rsqrt(1.0 + d0_ref[...] + d1_ref[...])
    dinv_ref[...] = dinv
    hp0_ref[...] = x_ref[...] * dinv


def _scale0(d0, d1, x):
    return pl.pallas_call(
        _scale0_body,
        grid=(NP // R,),
        in_specs=[_row_spec(1), _row_spec(1), _row_spec(128)],
        out_specs=[_row_spec(1), _row_spec(128)],
        out_shape=[
            jax.ShapeDtypeStruct((NP, 1), jnp.float32),
            jax.ShapeDtypeStruct((NP, 128), jnp.float32),
        ],
    )(d0, d1, x)


def _layer1_body(s0_ref, s1_ref, hp0_ref, dinv_ref, w1_ref, b1_ref, hp1_ref):
    dinv = dinv_ref[...]
    px = dinv * (s0_ref[...] + s1_ref[...] - hp0_ref[...])
    h1 = jnp.maximum(
        jnp.dot(px, w1_ref[...], preferred_element_type=jnp.float32)
        + b1_ref[...], 0.0)
    hp1_ref[...] = dinv * h1


def _layer1(s0, s1, hp0, dinv, W1, b1):
    return pl.pallas_call(
        _layer1_body,
        grid=(NP // R,),
        in_specs=[_row_spec(128), _row_spec(128), _row_spec(128), _row_spec(1),
                  _full_spec((128, 128)), _full_spec((1, 128))],
        out_specs=_row_spec(128),
        out_shape=jax.ShapeDtypeStruct((NP, 128), jnp.float32),
    )(s0, s1, hp0, dinv, W1, b1)


def _layer2_body(s0_ref, s1_ref, hp1_ref, dinv_ref, w2_ref, b2_ref, w3_ref,
                 gp_ref):
    dinv = dinv_ref[...]
    ph1 = dinv * (s0_ref[...] + s1_ref[...] - hp1_ref[...])
    h2 = jnp.maximum(
        jnp.dot(ph1, w2_ref[...], preferred_element_type=jnp.float32)
        + b2_ref[...], 0.0)
    gp_ref[...] = dinv * jnp.dot(h2, w3_ref[...],
                                 preferred_element_type=jnp.float32)


def _layer2(s0, s1, hp1, dinv, W2, b2, W3):
    return pl.pallas_call(
        _layer2_body,
        grid=(NP // R,),
        in_specs=[_row_spec(128), _row_spec(128), _row_spec(128), _row_spec(1),
                  _full_spec((128, 256)), _full_spec((1, 256)),
                  _full_spec((256, 64))],
        out_specs=_row_spec(64),
        out_shape=jax.ShapeDtypeStruct((NP, 64), jnp.float32),
    )(s0, s1, hp1, dinv, W2, b2, W3)


def _layer3_body(s0_ref, s1_ref, gp_ref, dinv_ref, b3_ref, out_ref):
    out_ref[...] = (dinv_ref[...] * (s0_ref[...] + s1_ref[...] - gp_ref[...])
                    + b3_ref[...])


def _layer3(s0, s1, gp, dinv, b3):
    return pl.pallas_call(
        _layer3_body,
        grid=(NP // R,),
        in_specs=[_row_spec(64), _row_spec(64), _row_spec(64), _row_spec(1),
                  _full_spec((1, 64))],
        out_specs=_row_spec(64),
        out_shape=jax.ShapeDtypeStruct((NP, 64), jnp.float32),
    )(s0, s1, gp, dinv, b3)


def kernel(x, edge_index, W1, b1, W2, b2, W3, b3):
    pad = N + (jnp.arange(EPAD - E, dtype=jnp.int32) % (NP - N))
    src = jnp.concatenate(
        [edge_index[0].astype(jnp.int32), pad]).reshape(NW * NCH, CH)
    dst = jnp.concatenate(
        [edge_index[1].astype(jnp.int32), pad]).reshape(NW * NCH, CH)
    xp = jnp.pad(x, ((0, NP - N), (0, 0)))

    d0, d1 = _deg(dst)
    dinv, hp0 = _scale0(d0.reshape(NP, 1), d1.reshape(NP, 1), xp)

    s0, s1 = _prop128(hp0, src, dst)
    hp1 = _layer1(s0, s1, hp0, dinv, W1, b1.reshape(1, 128))

    s0, s1 = _prop128(hp1, src, dst)
    gp = _layer2(s0, s1, hp1, dinv, W2, b2.reshape(1, 256), W3)

    s0, s1 = _prop64(gp, src, dst)
    out = _layer3(s0, s1, gp, dinv, b3.reshape(1, 64))
    return out[:N]


# speedup vs baseline: 1.0529x; 1.0529x over previous
import functools

import jax
import jax.numpy as jnp
from jax import lax
from jax.experimental import pallas as pl
from jax.experimental.pallas import tpu as pltpu
from jax.experimental.pallas import tpu_sc as plsc

N = 10000
NP = 10240
E = 320000
NC = 2
NS = 16
NW = NC * NS
CH = 128
EPAD = 327680
EPW = EPAD // NW
NCH = EPW // CH
RPT = NP // NS
ICH = 128

_MESH = plsc.VectorSubcoreMesh(core_axis_name="c", subcore_axis_name="s")


def _make_deg():
    @functools.partial(
        pl.kernel,
        out_type=[
            jax.ShapeDtypeStruct((NP,), jnp.float32),
            jax.ShapeDtypeStruct((NP,), jnp.float32),
        ],
        mesh=_MESH,
        scratch_types=[
            pltpu.VMEM((NCH, CH), jnp.int32),
            pltpu.VMEM((RPT,), jnp.float32),
            pltpu.VMEM((CH,), jnp.float32),
            pltpu.VMEM_SHARED((NP,), jnp.float32),
        ] + [pltpu.SemaphoreType.DMA] * 8,
    )
    def deg_kernel(dst_hbm, out0, out1, dst_v, zb_v, ones_v, acc_sh, *sems):
        cid = lax.axis_index("c")
        sid = lax.axis_index("s")
        w = sid * NC + cid
        r0 = sid * RPT

        pltpu.sync_copy(dst_hbm.at[pl.ds(w * NCH, NCH)], dst_v)

        def zfill(i, _):
            zb_v[pl.ds(i * 16, 16)] = jnp.zeros((16,), jnp.float32)
            return _
        lax.fori_loop(0, RPT // 16, zfill, 0)
        pltpu.sync_copy(zb_v, acc_sh.at[pl.ds(r0, RPT)])

        def ofill(i, _):
            ones_v[pl.ds(i * 16, 16)] = jnp.ones((16,), jnp.float32)
            return _
        lax.fori_loop(0, CH // 16, ofill, 0)
        plsc.subcore_barrier()

        NSEM = 8

        def body(jj, carry):
            j = jj * NSEM
            for b in range(NSEM):
                @pl.when(jj > 0)
                def _drain():
                    pltpu.make_async_copy(
                        ones_v, acc_sh.at[dst_v.at[0]], sems[b]).wait()
                pltpu.async_copy(ones_v, acc_sh.at[dst_v.at[j + b]],
                                 sems[b], add=True)
            return carry
        lax.fori_loop(0, NCH // NSEM, body, 0)
        for b in range(NSEM):
            pltpu.make_async_copy(ones_v, acc_sh.at[dst_v.at[0]],
                                  sems[b]).wait()
        plsc.subcore_barrier()

        pltpu.sync_copy(acc_sh.at[pl.ds(r0, RPT)], zb_v)

        @pl.when(cid == 0)
        def _():
            pltpu.sync_copy(zb_v, out0.at[pl.ds(r0, RPT)])

        @pl.when(cid == 1)
        def _():
            pltpu.sync_copy(zb_v, out1.at[pl.ds(r0, RPT)])

    return deg_kernel


def _make_prop(F, ch, nbuf):
    nch = EPW // ch
    hn = nch // 2
    assert hn % nbuf == 0 and RPT % ch == 0

    @functools.partial(
        pl.kernel,
        out_type=[
            jax.ShapeDtypeStruct((NP, F), jnp.float32),
            jax.ShapeDtypeStruct((NP, F), jnp.float32),
        ],
        mesh=_MESH,
        compiler_params=pltpu.CompilerParams(use_tc_tiling_on_sc=False),
        scratch_types=[
            pltpu.VMEM((hn, ch), jnp.int32),
            pltpu.VMEM((hn, ch), jnp.int32),
            pltpu.VMEM_SHARED((NP, F), jnp.float32),
        ] + [pltpu.VMEM((ch, F), jnp.float32)] * nbuf
          + [pltpu.SemaphoreType.DMA] * (2 * nbuf),
    )
    def prop_kernel(hp_hbm, src_hbm, dst_hbm, out0, out1,
                    src_v, dst_v, acc_sh, *scr):
        rows = scr[:nbuf]
        gsem = scr[nbuf:2 * nbuf]
        ssem = scr[2 * nbuf:]
        cid = lax.axis_index("c")
        sid = lax.axis_index("s")
        w = sid * NC + cid
        r0 = sid * RPT

        def wait_g(b):
            pltpu.make_async_copy(
                hp_hbm.at[src_v.at[0]], rows[b], gsem[b]).wait()

        def wait_s(b):
            pltpu.make_async_copy(
                rows[b], acc_sh.at[dst_v.at[0]], ssem[b]).wait()

        for t in range(RPT // ch):
            r = r0 + t * ch
            pltpu.sync_copy(hp_hbm.at[pl.ds(r, ch)], rows[0])
            pltpu.sync_copy(rows[0], acc_sh.at[pl.ds(r, ch)])
        plsc.subcore_barrier()

        for h in range(2):
            pltpu.sync_copy(src_hbm.at[pl.ds(w * nch + h * hn, hn)], src_v)
            pltpu.sync_copy(dst_hbm.at[pl.ds(w * nch + h * hn, hn)], dst_v)
            for b in range(nbuf - 1):
                pltpu.async_copy(hp_hbm.at[src_v.at[b]], rows[b], gsem[b])

            def body(jj, carry):
                j = jj * nbuf
                for b in range(nbuf):
                    p = j + b + nbuf - 1
                    pb = (b + nbuf - 1) % nbuf
                    if b == 0:
                        @pl.when(jj > 0)
                        def _w():
                            wait_s(pb)
                        pltpu.async_copy(
                            hp_hbm.at[src_v.at[p]], rows[pb], gsem[pb])
                    else:
                        @pl.when(p < hn)
                        def _gi():
                            wait_s(pb)
                            pltpu.async_copy(
                                hp_hbm.at[src_v.at[p]], rows[pb], gsem[pb])
                    wait_g(b)
                    pltpu.async_copy(rows[b], acc_sh.at[dst_v.at[j + b]],
                                     ssem[b], add=True)
                return carry
            lax.fori_loop(0, hn // nbuf, body, 0)
            for b in range(nbuf):
                wait_s(b)
        plsc.subcore_barrier()

        for t in range(RPT // ch):
            r = r0 + t * ch
            pltpu.sync_copy(acc_sh.at[pl.ds(r, ch)], rows[0])

            @pl.when(cid == 0)
            def _():
                pltpu.sync_copy(rows[0], out0.at[pl.ds(r, ch)])

            @pl.when(cid == 1)
            def _():
                pltpu.sync_copy(rows[0], out1.at[pl.ds(r, ch)])

    return prop_kernel


_deg = _make_deg()
_prop128 = _make_prop(128, 64, 4)
_prop64 = _make_prop(64, 128, 4)


R = 2048


def _row_spec(f):
    return pl.BlockSpec((R, f), lambda i: (i, 0))


def _full_spec(shape):
    return pl.BlockSpec(shape, lambda i: tuple(0 for _ in shape))


def _scale0_body(d0_ref, d1_ref, x_ref, dinv_ref, hp0_ref):
    dinv = lax.rsqrt(1.0 + d0_ref[...] + d1_ref[...])
    dinv_ref[...] = dinv
    hp0_ref[...] = x_ref[...] * dinv


def _scale0(d0, d1, x):
    return pl.pallas_call(
        _scale0_body,
        grid=(NP // R,),
        in_specs=[_row_spec(1), _row_spec(1), _row_spec(128)],
        out_specs=[_row_spec(1), _row_spec(128)],
        out_shape=[
            jax.ShapeDtypeStruct((NP, 1), jnp.float32),
            jax.ShapeDtypeStruct((NP, 128), jnp.float32),
        ],
    )(d0, d1, x)


def _layer1_body(s0_ref, s1_ref, hp0_ref, dinv_ref, w1_ref, b1_ref, hp1_ref):
    dinv = dinv_ref[...]
    px = dinv * (s0_ref[...] + s1_ref[...] - hp0_ref[...])
    h1 = jnp.maximum(
        jnp.dot(px, w1_ref[...], preferred_element_type=jnp.float32)
        + b1_ref[...], 0.0)
    hp1_ref[...] = dinv * h1


def _layer1(s0, s1, hp0, dinv, W1, b1):
    return pl.pallas_call(
        _layer1_body,
        grid=(NP // R,),
        in_specs=[_row_spec(128), _row_spec(128), _row_spec(128), _row_spec(1),
                  _full_spec((128, 128)), _full_spec((1, 128))],
        out_specs=_row_spec(128),
        out_shape=jax.ShapeDtypeStruct((NP, 128), jnp.float32),
    )(s0, s1, hp0, dinv, W1, b1)


def _layer2_body(s0_ref, s1_ref, hp1_ref, dinv_ref, w2_ref, b2_ref, w3_ref,
                 gp_ref):
    dinv = dinv_ref[...]
    ph1 = dinv * (s0_ref[...] + s1_ref[...] - hp1_ref[...])
    h2 = jnp.maximum(
        jnp.dot(ph1, w2_ref[...], preferred_element_type=jnp.float32)
        + b2_ref[...], 0.0)
    gp_ref[...] = dinv * jnp.dot(h2, w3_ref[...],
                                 preferred_element_type=jnp.float32)


def _layer2(s0, s1, hp1, dinv, W2, b2, W3):
    return pl.pallas_call(
        _layer2_body,
        grid=(NP // R,),
        in_specs=[_row_spec(128), _row_spec(128), _row_spec(128), _row_spec(1),
                  _full_spec((128, 256)), _full_spec((1, 256)),
                  _full_spec((256, 64))],
        out_specs=_row_spec(64),
        out_shape=jax.ShapeDtypeStruct((NP, 64), jnp.float32),
    )(s0, s1, hp1, dinv, W2, b2, W3)


def _layer3_body(s0_ref, s1_ref, gp_ref, dinv_ref, b3_ref, out_ref):
    out_ref[...] = (dinv_ref[...] * (s0_ref[...] + s1_ref[...] - gp_ref[...])
                    + b3_ref[...])


def _layer3(s0, s1, gp, dinv, b3):
    return pl.pallas_call(
        _layer3_body,
        grid=(NP // R,),
        in_specs=[_row_spec(64), _row_spec(64), _row_spec(64), _row_spec(1),
                  _full_spec((1, 64))],
        out_specs=_row_spec(64),
        out_shape=jax.ShapeDtypeStruct((NP, 64), jnp.float32),
    )(s0, s1, gp, dinv, b3)


def kernel(x, edge_index, W1, b1, W2, b2, W3, b3):
    pad = N + (jnp.arange(EPAD - E, dtype=jnp.int32) % (NP - N))
    src = jnp.concatenate([edge_index[0].astype(jnp.int32), pad])
    dst = jnp.concatenate([edge_index[1].astype(jnp.int32), pad])
    src64 = src.reshape(NW * (EPW // 64), 64)
    dst64 = dst.reshape(NW * (EPW // 64), 64)
    src128 = src.reshape(NW * NCH, CH)
    dst128 = dst.reshape(NW * NCH, CH)
    xp = jnp.pad(x, ((0, NP - N), (0, 0)))

    d0, d1 = _deg(dst128)
    dinv, hp0 = _scale0(d0.reshape(NP, 1), d1.reshape(NP, 1), xp)

    s0, s1 = _prop128(hp0, src64, dst64)
    hp1 = _layer1(s0, s1, hp0, dinv, W1, b1.reshape(1, 128))

    s0, s1 = _prop128(hp1, src64, dst64)
    gp = _layer2(s0, s1, hp1, dinv, W2, b2.reshape(1, 256), W3)

    s0, s1 = _prop64(gp, src128, dst128)
    out = _layer3(s0, s1, gp, dinv, b3.reshape(1, 64))
    return out[:N]


# speedup vs baseline: 1.1114x; 1.0556x over previous
import functools

import jax
import jax.numpy as jnp
from jax import lax
from jax.experimental import pallas as pl
from jax.experimental.pallas import tpu as pltpu
from jax.experimental.pallas import tpu_sc as plsc

N = 10000
NP = 10240
E = 320000
NC = 2
NS = 16
NW = NC * NS
CH = 128
EPAD = 327680
EPW = EPAD // NW
NCH = EPW // CH
RPT = NP // NS
ICH = 128

_MESH = plsc.VectorSubcoreMesh(core_axis_name="c", subcore_axis_name="s")


def _make_deg():
    @functools.partial(
        pl.kernel,
        out_type=[
            jax.ShapeDtypeStruct((NP,), jnp.float32),
            jax.ShapeDtypeStruct((NP,), jnp.float32),
        ],
        mesh=_MESH,
        scratch_types=[
            pltpu.VMEM((NCH, CH), jnp.int32),
            pltpu.VMEM((RPT,), jnp.float32),
            pltpu.VMEM((CH,), jnp.float32),
            pltpu.VMEM_SHARED((NP,), jnp.float32),
        ] + [pltpu.SemaphoreType.DMA] * 8,
    )
    def deg_kernel(dst_hbm, out0, out1, dst_v, zb_v, ones_v, acc_sh, *sems):
        cid = lax.axis_index("c")
        sid = lax.axis_index("s")
        w = sid * NC + cid
        r0 = sid * RPT

        pltpu.sync_copy(dst_hbm.at[pl.ds(w * NCH, NCH)], dst_v)

        def zfill(i, _):
            zb_v[pl.ds(i * 16, 16)] = jnp.zeros((16,), jnp.float32)
            return _
        lax.fori_loop(0, RPT // 16, zfill, 0)
        pltpu.sync_copy(zb_v, acc_sh.at[pl.ds(r0, RPT)])

        def ofill(i, _):
            ones_v[pl.ds(i * 16, 16)] = jnp.ones((16,), jnp.float32)
            return _
        lax.fori_loop(0, CH // 16, ofill, 0)
        plsc.subcore_barrier()

        NSEM = 8

        def body(jj, carry):
            j = jj * NSEM
            for b in range(NSEM):
                @pl.when(jj > 0)
                def _drain():
                    pltpu.make_async_copy(
                        ones_v, acc_sh.at[dst_v.at[0]], sems[b]).wait()
                pltpu.async_copy(ones_v, acc_sh.at[dst_v.at[j + b]],
                                 sems[b], add=True)
            return carry
        lax.fori_loop(0, NCH // NSEM, body, 0)
        for b in range(NSEM):
            pltpu.make_async_copy(ones_v, acc_sh.at[dst_v.at[0]],
                                  sems[b]).wait()
        plsc.subcore_barrier()

        pltpu.sync_copy(acc_sh.at[pl.ds(r0, RPT)], zb_v)

        @pl.when(cid == 0)
        def _():
            pltpu.sync_copy(zb_v, out0.at[pl.ds(r0, RPT)])

        @pl.when(cid == 1)
        def _():
            pltpu.sync_copy(zb_v, out1.at[pl.ds(r0, RPT)])

    return deg_kernel


def _make_prop(F, ch, nbuf):
    nch = EPW // ch
    hn = nch // 2
    assert hn % nbuf == 0 and RPT % ch == 0

    @functools.partial(
        pl.kernel,
        out_type=[
            jax.ShapeDtypeStruct((NP, F), jnp.float32),
            jax.ShapeDtypeStruct((NP, F), jnp.float32),
        ],
        mesh=_MESH,
        compiler_params=pltpu.CompilerParams(use_tc_tiling_on_sc=False),
        scratch_types=[
            pltpu.VMEM((hn, ch), jnp.int32),
            pltpu.VMEM((hn, ch), jnp.int32),
            pltpu.VMEM_SHARED((NP, F), jnp.float32),
        ] + [pltpu.VMEM((ch, F), jnp.float32)] * nbuf
          + [pltpu.SemaphoreType.DMA] * (2 * nbuf),
    )
    def prop_kernel(hp_hbm, src_hbm, dst_hbm, out0, out1,
                    src_v, dst_v, acc_sh, *scr):
        rows = scr[:nbuf]
        gsem = scr[nbuf:2 * nbuf]
        ssem = scr[2 * nbuf:]
        cid = lax.axis_index("c")
        sid = lax.axis_index("s")
        w = sid * NC + cid
        r0 = sid * RPT

        def wait_g(b):
            pltpu.make_async_copy(
                hp_hbm.at[src_v.at[0]], rows[b], gsem[b]).wait()

        def wait_s(b):
            pltpu.make_async_copy(
                rows[b], acc_sh.at[dst_v.at[0]], ssem[b]).wait()

        def zloop(i, carry):
            q = i // (F // 16)
            f = (i % (F // 16)) * 16
            rows[0][q, pl.ds(f, 16)] = jnp.zeros((16,), jnp.float32)
            return carry
        lax.fori_loop(0, ch * (F // 16), zloop, 0)
        for t in range(RPT // ch):
            pltpu.sync_copy(rows[0], acc_sh.at[pl.ds(r0 + t * ch, ch)])
        plsc.subcore_barrier()

        for h in range(2):
            pltpu.sync_copy(src_hbm.at[pl.ds(w * nch + h * hn, hn)], src_v)
            pltpu.sync_copy(dst_hbm.at[pl.ds(w * nch + h * hn, hn)], dst_v)
            for b in range(nbuf - 1):
                pltpu.async_copy(hp_hbm.at[src_v.at[b]], rows[b], gsem[b])

            def body(jj, carry):
                j = jj * nbuf
                for b in range(nbuf):
                    p = j + b + nbuf - 1
                    pb = (b + nbuf - 1) % nbuf
                    if b == 0:
                        @pl.when(jj > 0)
                        def _w():
                            wait_s(pb)
                        pltpu.async_copy(
                            hp_hbm.at[src_v.at[p]], rows[pb], gsem[pb])
                    else:
                        @pl.when(p < hn)
                        def _gi():
                            wait_s(pb)
                            pltpu.async_copy(
                                hp_hbm.at[src_v.at[p]], rows[pb], gsem[pb])
                    wait_g(b)
                    pltpu.async_copy(rows[b], acc_sh.at[dst_v.at[j + b]],
                                     ssem[b], add=True)
                return carry
            lax.fori_loop(0, hn // nbuf, body, 0)
            for b in range(nbuf):
                wait_s(b)
        plsc.subcore_barrier()

        for t in range(RPT // ch):
            r = r0 + t * ch
            pltpu.sync_copy(acc_sh.at[pl.ds(r, ch)], rows[0])

            @pl.when(cid == 0)
            def _():
                pltpu.sync_copy(rows[0], out0.at[pl.ds(r, ch)])

            @pl.when(cid == 1)
            def _():
                pltpu.sync_copy(rows[0], out1.at[pl.ds(r, ch)])

    return prop_kernel


_deg = _make_deg()
_prop128 = _make_prop(128, 64, 4)
_prop64 = _make_prop(64, 128, 4)


R = 2048


def _row_spec(f):
    return pl.BlockSpec((R, f), lambda i: (i, 0))


def _full_spec(shape):
    return pl.BlockSpec(shape, lambda i: tuple(0 for _ in shape))


def _scale0_body(d0_ref, d1_ref, x_ref, dinv_ref, hp0_ref):
    dinv = lax.rsqrt(1.0 + d0_ref[...] + d1_ref[...])
    dinv_ref[...] = dinv
    hp0_ref[...] = x_ref[...] * dinv


def _scale0(d0, d1, x):
    return pl.pallas_call(
        _scale0_body,
        grid=(NP // R,),
        in_specs=[_row_spec(1), _row_spec(1), _row_spec(128)],
        out_specs=[_row_spec(1), _row_spec(128)],
        out_shape=[
            jax.ShapeDtypeStruct((NP, 1), jnp.float32),
            jax.ShapeDtypeStruct((NP, 128), jnp.float32),
        ],
    )(d0, d1, x)


def _layer1_body(s0_ref, s1_ref, hp0_ref, dinv_ref, w1_ref, b1_ref, hp1_ref):
    dinv = dinv_ref[...]
    px = dinv * (s0_ref[...] + s1_ref[...] + hp0_ref[...])
    h1 = jnp.maximum(
        jnp.dot(px, w1_ref[...], preferred_element_type=jnp.float32)
        + b1_ref[...], 0.0)
    hp1_ref[...] = dinv * h1


def _layer1(s0, s1, hp0, dinv, W1, b1):
    return pl.pallas_call(
        _layer1_body,
        grid=(NP // R,),
        in_specs=[_row_spec(128), _row_spec(128), _row_spec(128), _row_spec(1),
                  _full_spec((128, 128)), _full_spec((1, 128))],
        out_specs=_row_spec(128),
        out_shape=jax.ShapeDtypeStruct((NP, 128), jnp.float32),
    )(s0, s1, hp0, dinv, W1, b1)


def _layer2_body(s0_ref, s1_ref, hp1_ref, dinv_ref, w2_ref, b2_ref, w3_ref,
                 gp_ref):
    dinv = dinv_ref[...]
    ph1 = dinv * (s0_ref[...] + s1_ref[...] + hp1_ref[...])
    h2 = jnp.maximum(
        jnp.dot(ph1, w2_ref[...], preferred_element_type=jnp.float32)
        + b2_ref[...], 0.0)
    gp_ref[...] = dinv * jnp.dot(h2, w3_ref[...],
                                 preferred_element_type=jnp.float32)


def _layer2(s0, s1, hp1, dinv, W2, b2, W3):
    return pl.pallas_call(
        _layer2_body,
        grid=(NP // R,),
        in_specs=[_row_spec(128), _row_spec(128), _row_spec(128), _row_spec(1),
                  _full_spec((128, 256)), _full_spec((1, 256)),
                  _full_spec((256, 64))],
        out_specs=_row_spec(64),
        out_shape=jax.ShapeDtypeStruct((NP, 64), jnp.float32),
    )(s0, s1, hp1, dinv, W2, b2, W3)


def _layer3_body(s0_ref, s1_ref, gp_ref, dinv_ref, b3_ref, out_ref):
    out_ref[...] = (dinv_ref[...] * (s0_ref[...] + s1_ref[...] + gp_ref[...])
                    + b3_ref[...])


RO = 2000


def _o_spec(f):
    return pl.BlockSpec((RO, f), lambda i: (i, 0))


def _layer3(s0, s1, gp, dinv, b3):
    return pl.pallas_call(
        _layer3_body,
        grid=(N // RO,),
        in_specs=[_o_spec(64), _o_spec(64), _o_spec(64), _o_spec(1),
                  _full_spec((1, 64))],
        out_specs=_o_spec(64),
        out_shape=jax.ShapeDtypeStruct((N, 64), jnp.float32),
    )(s0, s1, gp, dinv, b3)


def kernel(x, edge_index, W1, b1, W2, b2, W3, b3):
    pad = N + (jnp.arange(EPAD - E, dtype=jnp.int32) % (NP - N))
    src = jnp.concatenate([edge_index[0].astype(jnp.int32), pad])
    dst = jnp.concatenate([edge_index[1].astype(jnp.int32), pad])
    src64 = src.reshape(NW * (EPW // 64), 64)
    dst64 = dst.reshape(NW * (EPW // 64), 64)
    src128 = src.reshape(NW * NCH, CH)
    dst128 = dst.reshape(NW * NCH, CH)
    xp = jnp.pad(x, ((0, NP - N), (0, 0)))

    d0, d1 = _deg(dst128)
    dinv, hp0 = _scale0(d0.reshape(NP, 1), d1.reshape(NP, 1), xp)

    s0, s1 = _prop128(hp0, src64, dst64)
    hp1 = _layer1(s0, s1, hp0, dinv, W1, b1.reshape(1, 128))

    s0, s1 = _prop128(hp1, src64, dst64)
    gp = _layer2(s0, s1, hp1, dinv, W2, b2.reshape(1, 256), W3)

    s0, s1 = _prop64(gp, src128, dst128)
    return _layer3(s0, s1, gp, dinv, b3.reshape(1, 64))


# speedup vs baseline: 1.1145x; 1.0028x over previous
import functools

import jax
import jax.numpy as jnp
from jax import lax
from jax.experimental import pallas as pl
from jax.experimental.pallas import tpu as pltpu
from jax.experimental.pallas import tpu_sc as plsc

N = 10000
NP = 10240
E = 320000
NC = 2
NS = 16
NW = NC * NS
CH = 128
EPAD = 327680
EPW = EPAD // NW
NCH = EPW // CH
RPT = NP // NS
ICH = 128

_MESH = plsc.VectorSubcoreMesh(core_axis_name="c", subcore_axis_name="s")


def _make_deg():
    @functools.partial(
        pl.kernel,
        out_type=[
            jax.ShapeDtypeStruct((NP,), jnp.float32),
            jax.ShapeDtypeStruct((NP,), jnp.float32),
        ],
        mesh=_MESH,
        scratch_types=[
            pltpu.VMEM((NCH, CH), jnp.int32),
            pltpu.VMEM((RPT,), jnp.float32),
            pltpu.VMEM((CH,), jnp.float32),
            pltpu.VMEM_SHARED((NP,), jnp.float32),
        ] + [pltpu.SemaphoreType.DMA] * 8,
    )
    def deg_kernel(dst_hbm, out0, out1, dst_v, zb_v, ones_v, acc_sh, *sems):
        cid = lax.axis_index("c")
        sid = lax.axis_index("s")
        w = sid * NC + cid
        r0 = sid * RPT

        pltpu.sync_copy(dst_hbm.at[pl.ds(w * NCH, NCH)], dst_v)

        def zfill(i, _):
            zb_v[pl.ds(i * 16, 16)] = jnp.zeros((16,), jnp.float32)
            return _
        lax.fori_loop(0, RPT // 16, zfill, 0)
        pltpu.sync_copy(zb_v, acc_sh.at[pl.ds(r0, RPT)])

        def ofill(i, _):
            ones_v[pl.ds(i * 16, 16)] = jnp.ones((16,), jnp.float32)
            return _
        lax.fori_loop(0, CH // 16, ofill, 0)
        plsc.subcore_barrier()

        NSEM = 8

        def body(jj, carry):
            j = jj * NSEM
            for b in range(NSEM):
                @pl.when(jj > 0)
                def _drain():
                    pltpu.make_async_copy(
                        ones_v, acc_sh.at[dst_v.at[0]], sems[b]).wait()
                pltpu.async_copy(ones_v, acc_sh.at[dst_v.at[j + b]],
                                 sems[b], add=True)
            return carry
        lax.fori_loop(0, NCH // NSEM, body, 0)
        for b in range(NSEM):
            pltpu.make_async_copy(ones_v, acc_sh.at[dst_v.at[0]],
                                  sems[b]).wait()
        plsc.subcore_barrier()

        pltpu.sync_copy(acc_sh.at[pl.ds(r0, RPT)], zb_v)

        @pl.when(cid == 0)
        def _():
            pltpu.sync_copy(zb_v, out0.at[pl.ds(r0, RPT)])

        @pl.when(cid == 1)
        def _():
            pltpu.sync_copy(zb_v, out1.at[pl.ds(r0, RPT)])

    return deg_kernel


def _make_prop(F, ch, nbuf):
    nch = EPW // ch
    hn = nch // 2
    assert hn % nbuf == 0 and RPT % ch == 0

    @functools.partial(
        pl.kernel,
        out_type=[
            jax.ShapeDtypeStruct((NP, F), jnp.float32),
            jax.ShapeDtypeStruct((NP, F), jnp.float32),
        ],
        mesh=_MESH,
        compiler_params=pltpu.CompilerParams(use_tc_tiling_on_sc=False),
        scratch_types=[
            pltpu.VMEM((hn, ch), jnp.int32),
            pltpu.VMEM((hn, ch), jnp.int32),
            pltpu.VMEM_SHARED((NP, F), jnp.float32),
        ] + [pltpu.VMEM((ch, F), jnp.float32)] * nbuf
          + [pltpu.SemaphoreType.DMA] * (2 * nbuf),
    )
    def prop_kernel(hp_hbm, src_hbm, dst_hbm, out0, out1,
                    src_v, dst_v, acc_sh, *scr):
        rows = scr[:nbuf]
        gsem = scr[nbuf:2 * nbuf]
        ssem = scr[2 * nbuf:]
        cid = lax.axis_index("c")
        sid = lax.axis_index("s")
        w = sid * NC + cid
        r0 = sid * RPT

        def wait_g(b):
            pltpu.make_async_copy(
                hp_hbm.at[src_v.at[0]], rows[b], gsem[b]).wait()

        def wait_s(b):
            pltpu.make_async_copy(
                rows[b], acc_sh.at[dst_v.at[0]], ssem[b]).wait()

        def zloop(i, carry):
            q = i // (F // 16)
            f = (i % (F // 16)) * 16
            rows[0][q, pl.ds(f, 16)] = jnp.zeros((16,), jnp.float32)
            return carry
        lax.fori_loop(0, ch * (F // 16), zloop, 0)
        for t in range(RPT // ch):
            pltpu.sync_copy(rows[0], acc_sh.at[pl.ds(r0 + t * ch, ch)])
        plsc.subcore_barrier()

        for h in range(2):
            pltpu.sync_copy(src_hbm.at[pl.ds(w * nch + h * hn, hn)], src_v)
            pltpu.sync_copy(dst_hbm.at[pl.ds(w * nch + h * hn, hn)], dst_v)
            for b in range(nbuf - 1):
                pltpu.async_copy(hp_hbm.at[src_v.at[b]], rows[b], gsem[b])

            def body(jj, carry):
                j = jj * nbuf
                for b in range(nbuf):
                    p = j + b + nbuf - 1
                    pb = (b + nbuf - 1) % nbuf
                    if b == 0:
                        @pl.when(jj > 0)
                        def _w():
                            wait_s(pb)
                        pltpu.async_copy(
                            hp_hbm.at[src_v.at[p]], rows[pb], gsem[pb])
                    else:
                        @pl.when(p < hn)
                        def _gi():
                            wait_s(pb)
                            pltpu.async_copy(
                                hp_hbm.at[src_v.at[p]], rows[pb], gsem[pb])
                    wait_g(b)
                    pltpu.async_copy(rows[b], acc_sh.at[dst_v.at[j + b]],
                                     ssem[b], add=True)
                return carry
            lax.fori_loop(0, hn // nbuf, body, 0)
            for b in range(nbuf):
                wait_s(b)
        plsc.subcore_barrier()

        for t in range(RPT // ch):
            r = r0 + t * ch
            pltpu.sync_copy(acc_sh.at[pl.ds(r, ch)], rows[0])

            @pl.when(cid == 0)
            def _():
                pltpu.sync_copy(rows[0], out0.at[pl.ds(r, ch)])

            @pl.when(cid == 1)
            def _():
                pltpu.sync_copy(rows[0], out1.at[pl.ds(r, ch)])

    return prop_kernel


_deg = _make_deg()
_prop128 = _make_prop(128, 64, 4)
_prop64 = _make_prop(64, 128, 4)


R = 2048


def _row_spec(f):
    return pl.BlockSpec((R, f), lambda i: (i, 0))


def _full_spec(shape):
    return pl.BlockSpec(shape, lambda i: tuple(0 for _ in shape))


def _scale0_body(d0_ref, d1_ref, x_ref, dinv_ref, hp0_ref):
    dinv = lax.rsqrt(1.0 + d0_ref[...] + d1_ref[...])
    dinv_ref[...] = dinv
    hp0_ref[...] = x_ref[...] * dinv


def _scale0(d0, d1, x):
    return pl.pallas_call(
        _scale0_body,
        grid=(NP // R,),
        in_specs=[_row_spec(1), _row_spec(1), _row_spec(128)],
        out_specs=[_row_spec(1), _row_spec(128)],
        out_shape=[
            jax.ShapeDtypeStruct((NP, 1), jnp.float32),
            jax.ShapeDtypeStruct((NP, 128), jnp.float32),
        ],
    )(d0, d1, x)


def _layer1_body(s0_ref, s1_ref, hp0_ref, dinv_ref, w1_ref, b1_ref, hp1_ref):
    dinv = dinv_ref[...]
    px = dinv * (s0_ref[...] + s1_ref[...] + hp0_ref[...])
    h1 = jnp.maximum(
        jnp.dot(px, w1_ref[...], preferred_element_type=jnp.float32)
        + b1_ref[...], 0.0)
    hp1_ref[...] = dinv * h1


def _layer1(s0, s1, hp0, dinv, W1, b1):
    return pl.pallas_call(
        _layer1_body,
        grid=(NP // R,),
        in_specs=[_row_spec(128), _row_spec(128), _row_spec(128), _row_spec(1),
                  _full_spec((128, 128)), _full_spec((1, 128))],
        out_specs=_row_spec(128),
        out_shape=jax.ShapeDtypeStruct((NP, 128), jnp.float32),
    )(s0, s1, hp0, dinv, W1, b1)


def _layer2_body(s0_ref, s1_ref, hp1_ref, dinv_ref, w2_ref, b2_ref, w3_ref,
                 gp_ref):
    dinv = dinv_ref[...]
    ph1 = dinv * (s0_ref[...] + s1_ref[...] + hp1_ref[...])
    h2 = jnp.maximum(
        jnp.dot(ph1, w2_ref[...], preferred_element_type=jnp.float32)
        + b2_ref[...], 0.0)
    gp_ref[...] = dinv * jnp.dot(h2, w3_ref[...],
                                 preferred_element_type=jnp.float32)


def _layer2(s0, s1, hp1, dinv, W2, b2, W3):
    return pl.pallas_call(
        _layer2_body,
        grid=(NP // R,),
        in_specs=[_row_spec(128), _row_spec(128), _row_spec(128), _row_spec(1),
                  _full_spec((128, 256)), _full_spec((1, 256)),
                  _full_spec((256, 64))],
        out_specs=_row_spec(64),
        out_shape=jax.ShapeDtypeStruct((NP, 64), jnp.float32),
    )(s0, s1, hp1, dinv, W2, b2, W3)


def _layer3_body(s0_ref, s1_ref, gp_ref, dinv_ref, b3_ref, out_ref):
    out_ref[...] = (dinv_ref[...] * (s0_ref[...] + s1_ref[...] + gp_ref[...])
                    + b3_ref[...])


RO = 2000


def _o_spec(f):
    return pl.BlockSpec((RO, f), lambda i: (i, 0))


def _layer3(s0, s1, gp, dinv, b3):
    return pl.pallas_call(
        _layer3_body,
        grid=(N // RO,),
        in_specs=[_o_spec(64), _o_spec(64), _o_spec(64), _o_spec(1),
                  _full_spec((1, 64))],
        out_specs=_o_spec(64),
        out_shape=jax.ShapeDtypeStruct((N, 64), jnp.float32),
    )(s0, s1, gp, dinv, b3)


def kernel(x, edge_index, W1, b1, W2, b2, W3, b3):
    pad = N + (jnp.arange(EPAD - E, dtype=jnp.int32) % (NP - N))
    src = jnp.concatenate([edge_index[0].astype(jnp.int32), pad])
    dst = jnp.concatenate([edge_index[1].astype(jnp.int32), pad])
    src64 = src.reshape(NW * (EPW // 64), 64)
    dst64 = dst.reshape(NW * (EPW // 64), 64)
    src128 = src.reshape(NW * NCH, CH)
    dst128 = dst.reshape(NW * NCH, CH)
    d0, d1 = _deg(dst128)
    dinv, hp0 = _scale0(d0.reshape(NP, 1), d1.reshape(NP, 1), x)

    s0, s1 = _prop128(hp0, src64, dst64)
    hp1 = _layer1(s0, s1, hp0, dinv, W1, b1.reshape(1, 128))

    s0, s1 = _prop128(hp1, src64, dst64)
    gp = _layer2(s0, s1, hp1, dinv, W2, b2.reshape(1, 256), W3)

    s0, s1 = _prop64(gp, src128, dst128)
    return _layer3(s0, s1, gp, dinv, b3.reshape(1, 64))


# speedup vs baseline: 1.1243x; 1.0088x over previous
import functools

import jax
import jax.numpy as jnp
from jax import lax
from jax.experimental import pallas as pl
from jax.experimental.pallas import tpu as pltpu
from jax.experimental.pallas import tpu_sc as plsc

N = 10000
NP = 10240
E = 320000
NC = 2
NS = 16
NW = NC * NS
CH = 128
EPAD = 327680
EPW = EPAD // NW
NCH = EPW // CH
RPT = NP // NS
ICH = 128

_MESH = plsc.VectorSubcoreMesh(core_axis_name="c", subcore_axis_name="s")


def _make_deg():
    @functools.partial(
        pl.kernel,
        out_type=[
            jax.ShapeDtypeStruct((NP,), jnp.float32),
            jax.ShapeDtypeStruct((NP,), jnp.float32),
        ],
        mesh=_MESH,
        scratch_types=[
            pltpu.VMEM((NCH, CH), jnp.int32),
            pltpu.VMEM((RPT,), jnp.float32),
            pltpu.VMEM((CH,), jnp.float32),
            pltpu.VMEM_SHARED((NP,), jnp.float32),
        ] + [pltpu.SemaphoreType.DMA] * 8,
    )
    def deg_kernel(dst_hbm, out0, out1, dst_v, zb_v, ones_v, acc_sh, *sems):
        cid = lax.axis_index("c")
        sid = lax.axis_index("s")
        w = sid * NC + cid
        r0 = sid * RPT

        pltpu.sync_copy(dst_hbm.at[pl.ds(w * NCH, NCH)], dst_v)

        def zfill(i, _):
            zb_v[pl.ds(i * 16, 16)] = jnp.zeros((16,), jnp.float32)
            return _
        lax.fori_loop(0, RPT // 16, zfill, 0)
        pltpu.sync_copy(zb_v, acc_sh.at[pl.ds(r0, RPT)])

        def ofill(i, _):
            ones_v[pl.ds(i * 16, 16)] = jnp.ones((16,), jnp.float32)
            return _
        lax.fori_loop(0, CH // 16, ofill, 0)
        plsc.subcore_barrier()

        NSEM = 8

        def body(jj, carry):
            j = jj * NSEM
            for b in range(NSEM):
                @pl.when(jj > 0)
                def _drain():
                    pltpu.make_async_copy(
                        ones_v, acc_sh.at[dst_v.at[0]], sems[b]).wait()
                pltpu.async_copy(ones_v, acc_sh.at[dst_v.at[j + b]],
                                 sems[b], add=True)
            return carry
        lax.fori_loop(0, NCH // NSEM, body, 0)
        for b in range(NSEM):
            pltpu.make_async_copy(ones_v, acc_sh.at[dst_v.at[0]],
                                  sems[b]).wait()
        plsc.subcore_barrier()

        pltpu.sync_copy(acc_sh.at[pl.ds(r0, RPT)], zb_v)

        @pl.when(cid == 0)
        def _():
            pltpu.sync_copy(zb_v, out0.at[pl.ds(r0, RPT)])

        @pl.when(cid == 1)
        def _():
            pltpu.sync_copy(zb_v, out1.at[pl.ds(r0, RPT)])

    return deg_kernel


def _make_prop(F, ch, nbuf, nhalves):
    nch = EPW // ch
    hn = nch // nhalves
    assert hn % nbuf == 0 and RPT % ch == 0

    @functools.partial(
        pl.kernel,
        out_type=[
            jax.ShapeDtypeStruct((NP, F), jnp.float32),
            jax.ShapeDtypeStruct((NP, F), jnp.float32),
        ],
        mesh=_MESH,
        compiler_params=pltpu.CompilerParams(use_tc_tiling_on_sc=False),
        scratch_types=[
            pltpu.VMEM((hn, ch), jnp.int32),
            pltpu.VMEM((hn, ch), jnp.int32),
            pltpu.VMEM_SHARED((NP, F), jnp.float32),
        ] + [pltpu.VMEM((ch, F), jnp.float32)] * nbuf
          + [pltpu.SemaphoreType.DMA] * (2 * nbuf),
    )
    def prop_kernel(hp_hbm, src_hbm, dst_hbm, out0, out1,
                    src_v, dst_v, acc_sh, *scr):
        rows = scr[:nbuf]
        gsem = scr[nbuf:2 * nbuf]
        ssem = scr[2 * nbuf:]
        cid = lax.axis_index("c")
        sid = lax.axis_index("s")
        w = sid * NC + cid
        r0 = sid * RPT

        def wait_g(b):
            pltpu.make_async_copy(
                hp_hbm.at[src_v.at[0]], rows[b], gsem[b]).wait()

        def wait_s(b):
            pltpu.make_async_copy(
                rows[b], acc_sh.at[dst_v.at[0]], ssem[b]).wait()

        def zloop(i, carry):
            q = i // (F // 16)
            f = (i % (F // 16)) * 16
            rows[0][q, pl.ds(f, 16)] = jnp.zeros((16,), jnp.float32)
            return carry
        lax.fori_loop(0, ch * (F // 16), zloop, 0)
        for t in range(RPT // ch):
            pltpu.sync_copy(rows[0], acc_sh.at[pl.ds(r0 + t * ch, ch)])
        plsc.subcore_barrier()

        for h in range(nhalves):
            pltpu.sync_copy(src_hbm.at[pl.ds(w * nch + h * hn, hn)], src_v)
            pltpu.sync_copy(dst_hbm.at[pl.ds(w * nch + h * hn, hn)], dst_v)
            for b in range(nbuf - 1):
                pltpu.async_copy(hp_hbm.at[src_v.at[b]], rows[b], gsem[b])

            def body(jj, carry):
                j = jj * nbuf
                for b in range(nbuf):
                    p = j + b + nbuf - 1
                    pb = (b + nbuf - 1) % nbuf
                    if b == 0:
                        @pl.when(jj > 0)
                        def _w():
                            wait_s(pb)
                        pltpu.async_copy(
                            hp_hbm.at[src_v.at[p]], rows[pb], gsem[pb])
                    else:
                        @pl.when(p < hn)
                        def _gi():
                            wait_s(pb)
                            pltpu.async_copy(
                                hp_hbm.at[src_v.at[p]], rows[pb], gsem[pb])
                    wait_g(b)
                    pltpu.async_copy(rows[b], acc_sh.at[dst_v.at[j + b]],
                                     ssem[b], add=True)
                return carry
            lax.fori_loop(0, hn // nbuf, body, 0)
            for b in range(nbuf):
                wait_s(b)
        plsc.subcore_barrier()

        for t in range(RPT // ch):
            r = r0 + t * ch
            pltpu.sync_copy(acc_sh.at[pl.ds(r, ch)], rows[0])

            @pl.when(cid == 0)
            def _():
                pltpu.sync_copy(rows[0], out0.at[pl.ds(r, ch)])

            @pl.when(cid == 1)
            def _():
                pltpu.sync_copy(rows[0], out1.at[pl.ds(r, ch)])

    return prop_kernel


_deg = _make_deg()
_prop128 = _make_prop(128, 64, 4, 2)
_prop64 = _make_prop(64, 128, 4, 1)


R = 2048


def _row_spec(f):
    return pl.BlockSpec((R, f), lambda i: (i, 0))


def _full_spec(shape):
    return pl.BlockSpec(shape, lambda i: tuple(0 for _ in shape))


def _scale0_body(d0_ref, d1_ref, x_ref, dinv_ref, hp0_ref):
    dinv = lax.rsqrt(1.0 + d0_ref[...] + d1_ref[...])
    dinv_ref[...] = dinv
    hp0_ref[...] = x_ref[...] * dinv


def _scale0(d0, d1, x):
    return pl.pallas_call(
        _scale0_body,
        grid=(NP // R,),
        in_specs=[_row_spec(1), _row_spec(1), _row_spec(128)],
        out_specs=[_row_spec(1), _row_spec(128)],
        out_shape=[
            jax.ShapeDtypeStruct((NP, 1), jnp.float32),
            jax.ShapeDtypeStruct((NP, 128), jnp.float32),
        ],
    )(d0, d1, x)


def _layer1_body(s0_ref, s1_ref, hp0_ref, dinv_ref, w1_ref, b1_ref, hp1_ref):
    dinv = dinv_ref[...]
    px = dinv * (s0_ref[...] + s1_ref[...] + hp0_ref[...])
    h1 = jnp.maximum(
        jnp.dot(px, w1_ref[...], preferred_element_type=jnp.float32)
        + b1_ref[...], 0.0)
    hp1_ref[...] = dinv * h1


def _layer1(s0, s1, hp0, dinv, W1, b1):
    return pl.pallas_call(
        _layer1_body,
        grid=(NP // R,),
        in_specs=[_row_spec(128), _row_spec(128), _row_spec(128), _row_spec(1),
                  _full_spec((128, 128)), _full_spec((1, 128))],
        out_specs=_row_spec(128),
        out_shape=jax.ShapeDtypeStruct((NP, 128), jnp.float32),
    )(s0, s1, hp0, dinv, W1, b1)


def _layer2_body(s0_ref, s1_ref, hp1_ref, dinv_ref, w2_ref, b2_ref, w3_ref,
                 gp_ref):
    dinv = dinv_ref[...]
    ph1 = dinv * (s0_ref[...] + s1_ref[...] + hp1_ref[...])
    h2 = jnp.maximum(
        jnp.dot(ph1, w2_ref[...], preferred_element_type=jnp.float32)
        + b2_ref[...], 0.0)
    gp_ref[...] = dinv * jnp.dot(h2, w3_ref[...],
                                 preferred_element_type=jnp.float32)


def _layer2(s0, s1, hp1, dinv, W2, b2, W3):
    return pl.pallas_call(
        _layer2_body,
        grid=(NP // R,),
        in_specs=[_row_spec(128), _row_spec(128), _row_spec(128), _row_spec(1),
                  _full_spec((128, 256)), _full_spec((1, 256)),
                  _full_spec((256, 64))],
        out_specs=_row_spec(64),
        out_shape=jax.ShapeDtypeStruct((NP, 64), jnp.float32),
    )(s0, s1, hp1, dinv, W2, b2, W3)


def _layer3_body(s0_ref, s1_ref, gp_ref, dinv_ref, b3_ref, out_ref):
    out_ref[...] = (dinv_ref[...] * (s0_ref[...] + s1_ref[...] + gp_ref[...])
                    + b3_ref[...])


RO = 2000


def _o_spec(f):
    return pl.BlockSpec((RO, f), lambda i: (i, 0))


def _layer3(s0, s1, gp, dinv, b3):
    return pl.pallas_call(
        _layer3_body,
        grid=(N // RO,),
        in_specs=[_o_spec(64), _o_spec(64), _o_spec(64), _o_spec(1),
                  _full_spec((1, 64))],
        out_specs=_o_spec(64),
        out_shape=jax.ShapeDtypeStruct((N, 64), jnp.float32),
    )(s0, s1, gp, dinv, b3)


def kernel(x, edge_index, W1, b1, W2, b2, W3, b3):
    pad = N + (jnp.arange(EPAD - E, dtype=jnp.int32) % (NP - N))
    src = jnp.concatenate([edge_index[0].astype(jnp.int32), pad])
    dst = jnp.concatenate([edge_index[1].astype(jnp.int32), pad])
    src64 = src.reshape(NW * (EPW // 64), 64)
    dst64 = dst.reshape(NW * (EPW // 64), 64)
    src128 = src.reshape(NW * NCH, CH)
    dst128 = dst.reshape(NW * NCH, CH)
    d0, d1 = _deg(dst128)
    dinv, hp0 = _scale0(d0.reshape(NP, 1), d1.reshape(NP, 1), x)

    s0, s1 = _prop128(hp0, src64, dst64)
    hp1 = _layer1(s0, s1, hp0, dinv, W1, b1.reshape(1, 128))

    s0, s1 = _prop128(hp1, src64, dst64)
    gp = _layer2(s0, s1, hp1, dinv, W2, b2.reshape(1, 256), W3)

    s0, s1 = _prop64(gp, src128, dst128)
    return _layer3(s0, s1, gp, dinv, b3.reshape(1, 64))


# speedup vs baseline: 1.1248x; 1.0004x over previous
import functools

import jax
import jax.numpy as jnp
from jax import lax
from jax.experimental import pallas as pl
from jax.experimental.pallas import tpu as pltpu
from jax.experimental.pallas import tpu_sc as plsc

N = 10000
NP = 10240
E = 320000
NC = 2
NS = 16
NW = NC * NS
CH = 128
EPAD = 327680
EPW = EPAD // NW
NCH = EPW // CH
RPT = NP // NS

_MESH = plsc.VectorSubcoreMesh(core_axis_name="c", subcore_axis_name="s")


def _make_deg():
    @functools.partial(
        pl.kernel,
        out_type=[
            jax.ShapeDtypeStruct((NP,), jnp.float32),
            jax.ShapeDtypeStruct((NP,), jnp.float32),
        ],
        mesh=_MESH,
        scratch_types=[
            pltpu.VMEM((NCH, CH), jnp.int32),
            pltpu.VMEM((RPT,), jnp.float32),
            pltpu.VMEM((CH,), jnp.float32),
            pltpu.VMEM_SHARED((NP,), jnp.float32),
        ] + [pltpu.SemaphoreType.DMA] * 8,
    )
    def deg_kernel(dst_hbm, out0, out1, dst_v, zb_v, ones_v, acc_sh, *sems):
        cid = lax.axis_index("c")
        sid = lax.axis_index("s")
        w = sid * NC + cid
        r0 = sid * RPT

        pltpu.sync_copy(dst_hbm.at[pl.ds(w * NCH, NCH)], dst_v)

        def zfill(i, _):
            zb_v[pl.ds(i * 16, 16)] = jnp.zeros((16,), jnp.float32)
            return _
        lax.fori_loop(0, RPT // 16, zfill, 0)
        pltpu.sync_copy(zb_v, acc_sh.at[pl.ds(r0, RPT)])

        def ofill(i, _):
            ones_v[pl.ds(i * 16, 16)] = jnp.ones((16,), jnp.float32)
            return _
        lax.fori_loop(0, CH // 16, ofill, 0)
        plsc.subcore_barrier()

        NSEM = 8

        def body(jj, carry):
            j = jj * NSEM
            for b in range(NSEM):
                @pl.when(jj > 0)
                def _drain():
                    pltpu.make_async_copy(
                        ones_v, acc_sh.at[dst_v.at[0]], sems[b]).wait()
                pltpu.async_copy(ones_v, acc_sh.at[dst_v.at[j + b]],
                                 sems[b], add=True)
            return carry
        lax.fori_loop(0, NCH // NSEM, body, 0)
        for b in range(NSEM):
            pltpu.make_async_copy(ones_v, acc_sh.at[dst_v.at[0]],
                                  sems[b]).wait()
        plsc.subcore_barrier()

        pltpu.sync_copy(acc_sh.at[pl.ds(r0, RPT)], zb_v)

        @pl.when(cid == 0)
        def _():
            pltpu.sync_copy(zb_v, out0.at[pl.ds(r0, RPT)])

        @pl.when(cid == 1)
        def _():
            pltpu.sync_copy(zb_v, out1.at[pl.ds(r0, RPT)])

    return deg_kernel


def _make_prop(F, ch, nbuf, nhalves):
    nch = EPW // ch
    hn = nch // nhalves
    assert hn % nbuf == 0 and RPT % ch == 0

    @functools.partial(
        pl.kernel,
        out_type=[
            jax.ShapeDtypeStruct((NP, F), jnp.float32),
            jax.ShapeDtypeStruct((NP, F), jnp.float32),
        ],
        mesh=_MESH,
        compiler_params=pltpu.CompilerParams(use_tc_tiling_on_sc=False),
        scratch_types=[
            pltpu.VMEM((hn, ch), jnp.int32),
            pltpu.VMEM((hn, ch), jnp.int32),
            pltpu.VMEM_SHARED((NP, F), jnp.float32),
        ] + [pltpu.VMEM((ch, F), jnp.float32)] * nbuf
          + [pltpu.SemaphoreType.DMA] * (2 * nbuf),
    )
    def prop_kernel(hp_hbm, src_hbm, dst_hbm, out0, out1,
                    src_v, dst_v, acc_sh, *scr):
        rows = scr[:nbuf]
        gsem = scr[nbuf:2 * nbuf]
        ssem = scr[2 * nbuf:]
        cid = lax.axis_index("c")
        sid = lax.axis_index("s")
        w = sid * NC + cid
        r0 = sid * RPT

        def wait_g(b):
            pltpu.make_async_copy(
                hp_hbm.at[src_v.at[0]], rows[b], gsem[b]).wait()

        def wait_s(b):
            pltpu.make_async_copy(
                rows[b], acc_sh.at[dst_v.at[0]], ssem[b]).wait()

        def zloop(i, carry):
            q = i // (F // 16)
            f = (i % (F // 16)) * 16
            rows[0][q, pl.ds(f, 16)] = jnp.zeros((16,), jnp.float32)
            return carry
        lax.fori_loop(0, ch * (F // 16), zloop, 0)
        for t in range(RPT // ch):
            pltpu.sync_copy(rows[0], acc_sh.at[pl.ds(r0 + t * ch, ch)])
        plsc.subcore_barrier()

        for h in range(nhalves):
            pltpu.sync_copy(src_hbm.at[pl.ds(w * nch + h * hn, hn)], src_v)
            pltpu.sync_copy(dst_hbm.at[pl.ds(w * nch + h * hn, hn)], dst_v)
            for b in range(nbuf - 1):
                pltpu.async_copy(hp_hbm.at[src_v.at[b]], rows[b], gsem[b])

            def body(jj, carry):
                j = jj * nbuf
                for b in range(nbuf):
                    p = j + b + nbuf - 1
                    pb = (b + nbuf - 1) % nbuf
                    if b == 0:
                        @pl.when(jj > 0)
                        def _w():
                            wait_s(pb)
                        pltpu.async_copy(
                            hp_hbm.at[src_v.at[p]], rows[pb], gsem[pb])
                    else:
                        @pl.when(p < hn)
                        def _gi():
                            wait_s(pb)
                            pltpu.async_copy(
                                hp_hbm.at[src_v.at[p]], rows[pb], gsem[pb])
                    wait_g(b)
                    pltpu.async_copy(rows[b], acc_sh.at[dst_v.at[j + b]],
                                     ssem[b], add=True)
                return carry
            lax.fori_loop(0, hn // nbuf, body, 0)
            for b in range(nbuf):
                wait_s(b)
        plsc.subcore_barrier()

        for t in range(RPT // ch):
            r = r0 + t * ch
            pltpu.sync_copy(acc_sh.at[pl.ds(r, ch)], rows[0])

            @pl.when(cid == 0)
            def _():
                pltpu.sync_copy(rows[0], out0.at[pl.ds(r, ch)])

            @pl.when(cid == 1)
            def _():
                pltpu.sync_copy(rows[0], out1.at[pl.ds(r, ch)])

    return prop_kernel


_deg = _make_deg()
_prop128 = _make_prop(128, 64, 4, 2)
_prop64 = _make_prop(64, 128, 4, 1)


R = 2048


def _row_spec(f):
    return pl.BlockSpec((R, f), lambda i: (i, 0))


def _full_spec(shape):
    return pl.BlockSpec(shape, lambda i: tuple(0 for _ in shape))


def _scale0_body(d0_ref, d1_ref, x_ref, dinv_ref, hp0_ref):
    dinv = lax.rsqrt(1.0 + d0_ref[...] + d1_ref[...])
    dinv_ref[...] = dinv
    hp0_ref[...] = x_ref[...] * dinv


def _scale0(d0, d1, x):
    return pl.pallas_call(
        _scale0_body,
        grid=(NP // R,),
        in_specs=[_row_spec(1), _row_spec(1), _row_spec(128)],
        out_specs=[_row_spec(1), _row_spec(128)],
        out_shape=[
            jax.ShapeDtypeStruct((NP, 1), jnp.float32),
            jax.ShapeDtypeStruct((NP, 128), jnp.float32),
        ],
    )(d0, d1, x)


def _layer1_body(s0_ref, s1_ref, hp0_ref, dinv_ref, w1_ref, b1_ref, hp1_ref):
    dinv = dinv_ref[...]
    px = dinv * (s0_ref[...] + s1_ref[...] + hp0_ref[...])
    h1 = jnp.maximum(
        jnp.dot(px, w1_ref[...], preferred_element_type=jnp.float32)
        + b1_ref[...], 0.0)
    hp1_ref[...] = dinv * h1


def _layer1(s0, s1, hp0, dinv, W1, b1):
    return pl.pallas_call(
        _layer1_body,
        grid=(NP // R,),
        in_specs=[_row_spec(128), _row_spec(128), _row_spec(128), _row_spec(1),
                  _full_spec((128, 128)), _full_spec((1, 128))],
        out_specs=_row_spec(128),
        out_shape=jax.ShapeDtypeStruct((NP, 128), jnp.float32),
    )(s0, s1, hp0, dinv, W1, b1)


def _layer2_body(s0_ref, s1_ref, hp1_ref, dinv_ref, w2_ref, b2_ref, w3_ref,
                 gp_ref):
    dinv = dinv_ref[...]
    ph1 = dinv * (s0_ref[...] + s1_ref[...] + hp1_ref[...])
    h2 = jnp.maximum(
        jnp.dot(ph1, w2_ref[...], preferred_element_type=jnp.float32)
        + b2_ref[...], 0.0)
    gp_ref[...] = dinv * jnp.dot(h2, w3_ref[...],
                                 preferred_element_type=jnp.float32)


def _layer2(s0, s1, hp1, dinv, W2, b2, W3):
    return pl.pallas_call(
        _layer2_body,
        grid=(NP // R,),
        in_specs=[_row_spec(128), _row_spec(128), _row_spec(128), _row_spec(1),
                  _full_spec((128, 256)), _full_spec((1, 256)),
                  _full_spec((256, 64))],
        out_specs=_row_spec(64),
        out_shape=jax.ShapeDtypeStruct((NP, 64), jnp.float32),
    )(s0, s1, hp1, dinv, W2, b2, W3)


def _layer3_body(s0_ref, s1_ref, gp_ref, dinv_ref, b3_ref, out_ref):
    out_ref[...] = (dinv_ref[...] * (s0_ref[...] + s1_ref[...] + gp_ref[...])
                    + b3_ref[...])


RO = 2000


def _o_spec(f):
    return pl.BlockSpec((RO, f), lambda i: (i, 0))


def _layer3(s0, s1, gp, dinv, b3):
    return pl.pallas_call(
        _layer3_body,
        grid=(N // RO,),
        in_specs=[_o_spec(64), _o_spec(64), _o_spec(64), _o_spec(1),
                  _full_spec((1, 64))],
        out_specs=_o_spec(64),
        out_shape=jax.ShapeDtypeStruct((N, 64), jnp.float32),
    )(s0, s1, gp, dinv, b3)


def kernel(x, edge_index, W1, b1, W2, b2, W3, b3):
    pad = N + (jnp.arange(EPAD - E, dtype=jnp.int32) % (NP - N))
    src = jnp.concatenate([edge_index[0].astype(jnp.int32), pad])
    dst = jnp.concatenate([edge_index[1].astype(jnp.int32), pad])
    src64 = src.reshape(NW * (EPW // 64), 64)
    dst64 = dst.reshape(NW * (EPW // 64), 64)
    src128 = src.reshape(NW * NCH, CH)
    dst128 = dst.reshape(NW * NCH, CH)
    d0, d1 = _deg(dst128)
    dinv, hp0 = _scale0(d0.reshape(NP, 1), d1.reshape(NP, 1), x)

    s0, s1 = _prop128(hp0, src64, dst64)
    hp1 = _layer1(s0, s1, hp0, dinv, W1, b1.reshape(1, 128))

    s0, s1 = _prop128(hp1, src64, dst64)
    gp = _layer2(s0, s1, hp1, dinv, W2, b2.reshape(1, 256), W3)

    s0, s1 = _prop64(gp, src128, dst128)
    return _layer3(s0, s1, gp, dinv, b3.reshape(1, 64))
